# baseline (speedup 1.0000x reference)
import jax
import jax.numpy as jnp
from jax.experimental import pallas as pl
from jax.experimental.pallas import tpu as pltpu

_TILE_N = 512
_TILE_K = 2048
_CHUNK = 128


def _vq_assign_kernel(z_ref, cbt_ref, z2_ref, c2_ref, out_ref, val_ref):
    k = pl.program_id(1)

    s = jax.lax.dot_general(
        z_ref[...], cbt_ref[...],
        dimension_numbers=(((1,), (0,)), ((), ())),
        preferred_element_type=jnp.float32,
    )
    d = (z2_ref[...][:, None] - 2.0 * s) + c2_ref[...][None, :]

    nchunk = _TILE_K // _CHUNK
    tn = d.shape[0]
    d3 = d.reshape(tn, nchunk, _CHUNK)
    cmin = jnp.min(d3, axis=2)
    lane_iota = jax.lax.broadcasted_iota(jnp.int32, d3.shape, 2)
    big = jnp.int32(2**30)
    carg = jnp.min(jnp.where(d3 == cmin[:, :, None], lane_iota, big), axis=2)

    @pl.when(k == 0)
    def _init():
        val_ref[...] = jnp.full_like(val_ref, jnp.inf)
        out_ref[...] = jnp.zeros_like(out_ref)

    for c in range(nchunk):
        m = cmin[:, c]
        a = carg[:, c]
        col = a + (k * _TILE_K + c * _CHUNK)
        acc = m < val_ref[...]
        mb = m.astype(jnp.bfloat16).astype(jnp.float32)
        val_ref[...] = jnp.where(acc, mb, val_ref[...])
        out_ref[...] = jnp.where(acc, col, out_ref[...])


def kernel(z, codebook):
    n, dim = z.shape
    num_codes = codebook.shape[0]
    cbt = codebook.T
    z2 = jnp.sum(z * z, axis=1)
    c2 = jnp.sum(codebook * codebook, axis=1)
    grid = (n // _TILE_N, num_codes // _TILE_K)
    return pl.pallas_call(
        _vq_assign_kernel,
        grid=grid,
        in_specs=[
            pl.BlockSpec((_TILE_N, dim), lambda i, k: (i, 0)),
            pl.BlockSpec((dim, _TILE_K), lambda i, k: (0, k)),
            pl.BlockSpec((_TILE_N,), lambda i, k: (i,)),
            pl.BlockSpec((_TILE_K,), lambda i, k: (k,)),
        ],
        out_specs=pl.BlockSpec((_TILE_N,), lambda i, k: (i,)),
        out_shape=jax.ShapeDtypeStruct((n,), jnp.int32),
        scratch_shapes=[pltpu.VMEM((_TILE_N,), jnp.float32)],
        compiler_params=pltpu.CompilerParams(
            dimension_semantics=("parallel", "arbitrary"),
        ),
    )(z, cbt, z2, c2)


# speedup vs baseline: 1.8543x; 1.8543x over previous
import jax
import jax.numpy as jnp
from jax.experimental import pallas as pl
from jax.experimental.pallas import tpu as pltpu

_TILE_N = 1024
_TILE_K = 128


def _vq_assign_kernel(z_ref, cbt_ref, z2_ref, c2_ref, out_ref, val_ref):
    k = pl.program_id(1)

    s = jax.lax.dot_general(
        z_ref[...], cbt_ref[...],
        dimension_numbers=(((1,), (0,)), ((), ())),
        preferred_element_type=jnp.float32,
    )
    d = (z2_ref[...][:, None] - 2.0 * s) + c2_ref[...][None, :]

    m = jnp.min(d, axis=1)
    lane_iota = jax.lax.broadcasted_iota(jnp.int32, d.shape, 1)
    big = jnp.int32(2**30)
    a = jnp.min(jnp.where(d == m[:, None], lane_iota, big), axis=1)
    col = a + k * _TILE_K

    @pl.when(k == 0)
    def _init():
        val_ref[...] = jnp.full_like(val_ref, jnp.inf)
        out_ref[...] = jnp.zeros_like(out_ref)

    acc = m < val_ref[...]
    mb = m.astype(jnp.bfloat16).astype(jnp.float32)
    val_ref[...] = jnp.where(acc, mb, val_ref[...])
    out_ref[...] = jnp.where(acc, col, out_ref[...])


def kernel(z, codebook):
    n, dim = z.shape
    num_codes = codebook.shape[0]
    cbt = codebook.T
    z2 = jnp.sum(z * z, axis=1)
    c2 = jnp.sum(codebook * codebook, axis=1)
    grid = (n // _TILE_N, num_codes // _TILE_K)
    return pl.pallas_call(
        _vq_assign_kernel,
        grid=grid,
        in_specs=[
            pl.BlockSpec((_TILE_N, dim), lambda i, k: (i, 0)),
            pl.BlockSpec((dim, _TILE_K), lambda i, k: (0, k)),
            pl.BlockSpec((_TILE_N,), lambda i, k: (i,)),
            pl.BlockSpec((_TILE_K,), lambda i, k: (k,)),
        ],
        out_specs=pl.BlockSpec((_TILE_N,), lambda i, k: (i,)),
        out_shape=jax.ShapeDtypeStruct((n,), jnp.int32),
        scratch_shapes=[pltpu.VMEM((_TILE_N,), jnp.float32)],
        compiler_params=pltpu.CompilerParams(
            dimension_semantics=("parallel", "arbitrary"),
        ),
    )(z, cbt, z2, c2)
